# Initial kernel scaffold; baseline (speedup 1.0000x reference)
import jax
import jax.numpy as jnp
from jax.experimental import pallas as pl


def kernel(token_ids, section, temporality, negated, timestamp_bucket, token_table, section_table, temporality_table, negation_table, position_table, timestamp_table, ln_gamma, ln_beta, W, b):
    raise NotImplementedError("write your pallas kernel here")


# speedup vs baseline: 4.4281x; 4.4281x over previous
import functools

import jax
import jax.numpy as jnp
from jax import lax
from jax.experimental import pallas as pl
from jax.experimental.pallas import tpu as pltpu
from jax.experimental.pallas import tpu_sc as plsc

B, S, L, D = 1024, 50, 20, 64
V = 100000
TB = 512
BS = B * S

NC, NS = 2, 16
NW = NC * NS
COLS_W = BS // NW
CCH = 80
NCHUNK = COLS_W // CCH
NAUX = 5

R_TC = 3200
G_TC = BS // R_TC


def _sc_gather_sum(tid_t, aux_idx, token_table, aux_table):
  mesh = plsc.VectorSubcoreMesh(core_axis_name="c", subcore_axis_name="s")

  @functools.partial(
      pl.kernel,
      mesh=mesh,
      out_type=jax.ShapeDtypeStruct((BS, D), jnp.float32),
      scratch_types=[
          pltpu.VMEM((L, COLS_W), jnp.int32),
          pltpu.VMEM((NAUX, COLS_W), jnp.int32),
          pltpu.VMEM((CCH, D), jnp.float32),
          pltpu.SemaphoreType.DMA,
          pltpu.SemaphoreType.DMA,
      ],
      compiler_params=pltpu.CompilerParams(use_tc_tiling_on_sc=False),
  )
  def body(tid_hbm, aux_hbm, table_hbm, auxtab_hbm, out_hbm, idx_v, aidx_v,
           acc_v, sem, sem2):
    wid = lax.axis_index("s") * NC + lax.axis_index("c")
    base = wid * COLS_W
    for l in range(L):
      pltpu.sync_copy(tid_hbm.at[l, pl.ds(base, COLS_W)], idx_v.at[l])
    for a in range(NAUX):
      pltpu.sync_copy(aux_hbm.at[a, pl.ds(base, COLS_W)], aidx_v.at[a])

    def chunk(ci, carry):
      off = ci * CCH
      pltpu.async_copy(
          table_hbm.at[idx_v.at[0, pl.ds(off, CCH)]], acc_v, sem).wait()
      descs = []
      for l in range(1, L):
        descs.append(
            pltpu.async_copy(
                table_hbm.at[idx_v.at[l, pl.ds(off, CCH)]], acc_v, sem2,
                add=True))
      for a in range(NAUX):
        descs.append(
            pltpu.async_copy(
                auxtab_hbm.at[aidx_v.at[a, pl.ds(off, CCH)]], acc_v, sem2,
                add=True))
      for dsc in descs:
        dsc.wait()
      pltpu.sync_copy(acc_v, out_hbm.at[pl.ds(base + off, CCH)])
      return carry

    lax.fori_loop(0, NCHUNK, chunk, 0)

  return body(tid_t, aux_idx, token_table, aux_table)


def _tc_finish(pre, gamma2, beta2, W, b2):
  def body(pre_ref, g_ref, be_ref, w_ref, b_ref, o_ref):
    x = pre_ref[...] * (1.0 / L)
    mu = jnp.mean(x, axis=1, keepdims=True)
    xc = x - mu
    var = jnp.mean(xc * xc, axis=1, keepdims=True)
    nx = xc * lax.rsqrt(var + 1e-5) * g_ref[...] + be_ref[...]
    y = lax.dot_general(nx, w_ref[...], (((1,), (1,)), ((), ())),
                        preferred_element_type=jnp.float32,
                        precision=lax.Precision.HIGHEST)
    o_ref[...] = y + b_ref[...]

  return pl.pallas_call(
      body,
      grid=(G_TC,),
      in_specs=[
          pl.BlockSpec((R_TC, D), lambda i: (i, 0)),
          pl.BlockSpec((1, D), lambda i: (0, 0)),
          pl.BlockSpec((1, D), lambda i: (0, 0)),
          pl.BlockSpec((D, D), lambda i: (0, 0)),
          pl.BlockSpec((1, D), lambda i: (0, 0)),
      ],
      out_specs=pl.BlockSpec((R_TC, D), lambda i: (i, 0)),
      out_shape=jax.ShapeDtypeStruct((BS, D), jnp.float32),
  )(pre, gamma2, beta2, W, b2)


def kernel(token_ids, section, temporality, negated, timestamp_bucket,
           token_table, section_table, temporality_table, negation_table,
           position_table, timestamp_table, ln_gamma, ln_beta, W, b):
  tid_t = token_ids.astype(jnp.int32).reshape(BS, L).T

  aux_table = jnp.concatenate([
      section_table, temporality_table, negation_table, timestamp_table,
      position_table
  ], axis=0) * float(L)
  pos_idx = jnp.tile(jnp.arange(S, dtype=jnp.int32), B) + (6 + 3 + 2 + TB)
  aux_idx = jnp.stack([
      section.astype(jnp.int32).reshape(BS),
      temporality.astype(jnp.int32).reshape(BS) + 6,
      negated.astype(jnp.int32).reshape(BS) + 9,
      timestamp_bucket.astype(jnp.int32).reshape(BS) + 11,
      pos_idx,
  ])

  pre = _sc_gather_sum(tid_t, aux_idx, token_table, aux_table)
  tokens_flat = _tc_finish(pre, ln_gamma.reshape(1, D), ln_beta.reshape(1, D),
                           W, b.reshape(1, D))
  tokens = tokens_flat.reshape(B, S, D)
  padding_mask = jnp.zeros((B, S), dtype=bool)
  return tokens, padding_mask
